# Initial kernel scaffold; baseline (speedup 1.0000x reference)
import jax
import jax.numpy as jnp
from jax.experimental import pallas as pl


def kernel(x, edge_idx, W1, b1, W2, b2):
    raise NotImplementedError("write your pallas kernel here")


# speedup vs baseline: 14.2499x; 14.2499x over previous
import functools

import jax
import jax.numpy as jnp
from jax import lax
from jax.experimental import pallas as pl
from jax.experimental.pallas import tpu as pltpu, tpu_sc as plsc

N = 10000
E = 320000
DF = 128
DH = 256
DC = 16

NC = 2
NS = 16
ROW = 128
EPAD = 2560 * ROW
RPT = 2560 // (NC * NS)
NACC = 10240
NP = NACC

_MESH = plsc.VectorSubcoreMesh(core_axis_name="c", subcore_axis_name="s")
_SC_PARAMS = pltpu.CompilerParams(use_tc_tiling_on_sc=False)


def _sc_deg(dst2d, zeros16, ones16):
    @functools.partial(
        pl.kernel,
        out_type=jax.ShapeDtypeStruct((NC, NP, DC), jnp.float32),
        mesh=_MESH,
        compiler_params=_SC_PARAMS,
        scratch_types=[
            pltpu.VMEM((RPT, ROW), jnp.int32),
            pltpu.VMEM((ROW, DC), jnp.float32),
            pltpu.VMEM_SHARED((NACC, DC), jnp.float32),
        ],
    )
    def k(dst_hbm, z_hbm, ones_hbm, out_hbm, dstv, onev, acc):
        c = lax.axis_index("c")
        s = lax.axis_index("s")
        base = (c * NS + s) * RPT
        pltpu.sync_copy(z_hbm.at[pl.ds(s * (NACC // NS), NACC // NS)],
                        acc.at[pl.ds(s * (NACC // NS), NACC // NS)])
        pltpu.sync_copy(dst_hbm.at[pl.ds(base, RPT)], dstv)
        pltpu.sync_copy(ones_hbm, onev)
        plsc.subcore_barrier()

        def step(j, carry):
            pltpu.sync_copy(onev, acc.at[dstv.at[j]], add=True)
            return carry

        lax.fori_loop(0, RPT, step, 0)
        plsc.subcore_barrier()
        pltpu.sync_copy(acc.at[pl.ds(s * (NP // NS), NP // NS)],
                        out_hbm.at[c, pl.ds(s * (NP // NS), NP // NS)])

    return k(dst2d, zeros16, ones16)


def _sc_agg(table, src2d, dst2d, zeros, d):
    @functools.partial(
        pl.kernel,
        out_type=jax.ShapeDtypeStruct((NC, NP, d), jnp.float32),
        mesh=_MESH,
        compiler_params=_SC_PARAMS,
        scratch_types=[
            pltpu.VMEM((RPT, ROW), jnp.int32),
            pltpu.VMEM((RPT, ROW), jnp.int32),
            pltpu.VMEM((ROW, d), jnp.float32),
            pltpu.VMEM_SHARED((NACC, d), jnp.float32),
            pltpu.SemaphoreType.DMA,
        ],
    )
    def k(tab_hbm, src_hbm, dst_hbm, z_hbm, out_hbm, srcv, dstv, rows, acc, sem):
        c = lax.axis_index("c")
        s = lax.axis_index("s")
        base = (c * NS + s) * RPT
        pltpu.sync_copy(z_hbm.at[pl.ds(s * (NACC // NS), NACC // NS)],
                        acc.at[pl.ds(s * (NACC // NS), NACC // NS)])
        pltpu.sync_copy(src_hbm.at[pl.ds(base, RPT)], srcv)
        pltpu.sync_copy(dst_hbm.at[pl.ds(base, RPT)], dstv)
        plsc.subcore_barrier()

        def step(j, carry):
            pltpu.async_copy(tab_hbm.at[srcv.at[j]], rows, sem).wait()
            pltpu.sync_copy(rows, acc.at[dstv.at[j]], add=True)
            return carry

        lax.fori_loop(0, RPT, step, 0)
        plsc.subcore_barrier()
        pltpu.sync_copy(acc.at[pl.ds(s * (NP // NS), NP // NS)],
                        out_hbm.at[c, pl.ds(s * (NP // NS), NP // NS)])

    return k(table, src2d, dst2d, zeros)


_BLK = 1024
_GRID = NP // _BLK


def _tc_prep(degp, x):
    def body(degp_ref, x_ref, dinv_ref, xs_ref):
        deg = degp_ref[0, :, 0:1] + degp_ref[1, :, 0:1] + 1.0
        dinv = lax.rsqrt(deg)
        dinv_ref[...] = dinv
        xs_ref[...] = x_ref[...] * dinv

    return pl.pallas_call(
        body,
        grid=(_GRID,),
        in_specs=[
            pl.BlockSpec((NC, _BLK, DC), lambda i: (0, i, 0)),
            pl.BlockSpec((_BLK, DF), lambda i: (i, 0)),
        ],
        out_specs=[
            pl.BlockSpec((_BLK, 1), lambda i: (i, 0)),
            pl.BlockSpec((_BLK, DF), lambda i: (i, 0)),
        ],
        out_shape=[
            jax.ShapeDtypeStruct((NP, 1), jnp.float32),
            jax.ShapeDtypeStruct((NP, DF), jnp.float32),
        ],
    )(degp, x)


def _tc_mid(Sp, xs, dinv, W1, b1, W2):
    def body(S_ref, xs_ref, dinv_ref, W1_ref, b1_ref, W2_ref, zs_ref):
        dinv = dinv_ref[...]
        agg = (S_ref[0] + S_ref[1] + xs_ref[...]) * dinv
        h = jnp.dot(agg, W1_ref[...], preferred_element_type=jnp.float32)
        h = jnp.maximum(h + b1_ref[...], 0.0)
        z = jnp.dot(h, W2_ref[...], preferred_element_type=jnp.float32)
        zs_ref[...] = z * dinv

    return pl.pallas_call(
        body,
        grid=(_GRID,),
        in_specs=[
            pl.BlockSpec((NC, _BLK, DF), lambda i: (0, i, 0)),
            pl.BlockSpec((_BLK, DF), lambda i: (i, 0)),
            pl.BlockSpec((_BLK, 1), lambda i: (i, 0)),
            pl.BlockSpec((DF, DH), lambda i: (0, 0)),
            pl.BlockSpec((1, DH), lambda i: (0, 0)),
            pl.BlockSpec((DH, DC), lambda i: (0, 0)),
        ],
        out_specs=pl.BlockSpec((_BLK, DC), lambda i: (i, 0)),
        out_shape=jax.ShapeDtypeStruct((NP, DC), jnp.float32),
    )(Sp, xs, dinv, W1, b1, W2)


def _tc_final(Tp, zs, dinv, b2):
    def body(T_ref, zs_ref, dinv_ref, b2_ref, out_ref):
        t = (T_ref[0] + T_ref[1] + zs_ref[...]) * dinv_ref[...] + b2_ref[...]
        m = jnp.max(t, axis=1, keepdims=True)
        e = jnp.exp(t - m)
        lse = jnp.log(jnp.sum(e, axis=1, keepdims=True))
        out_ref[...] = t - m - lse

    return pl.pallas_call(
        body,
        grid=(_GRID,),
        in_specs=[
            pl.BlockSpec((NC, _BLK, DC), lambda i: (0, i, 0)),
            pl.BlockSpec((_BLK, DC), lambda i: (i, 0)),
            pl.BlockSpec((_BLK, 1), lambda i: (i, 0)),
            pl.BlockSpec((1, DC), lambda i: (0, 0)),
        ],
        out_specs=pl.BlockSpec((_BLK, DC), lambda i: (i, 0)),
        out_shape=jax.ShapeDtypeStruct((NP, DC), jnp.float32),
    )(Tp, zs, dinv, b2)


def kernel(x, edge_idx, W1, b1, W2, b2):
    src = edge_idx[0]
    dst = edge_idx[1]
    pad = EPAD - E
    src2d = jnp.concatenate([src, jnp.zeros((pad,), jnp.int32)]).reshape(-1, ROW)
    dst2d = jnp.concatenate([dst, jnp.full((pad,), N, jnp.int32)]).reshape(-1, ROW)

    zeros16 = jnp.zeros((NACC, DC), jnp.float32)
    zeros128 = jnp.zeros((NACC, DF), jnp.float32)
    ones16 = jnp.ones((ROW, DC), jnp.float32)

    xp = jnp.pad(x, ((0, NP - N), (0, 0)))
    degp = _sc_deg(dst2d, zeros16, ones16)
    dinv, xs = _tc_prep(degp, xp)
    Sp = _sc_agg(xs, src2d, dst2d, zeros128, DF)
    zs = _tc_mid(Sp, xs, dinv, W1, b1[None, :], W2)
    Tp = _sc_agg(zs, src2d, dst2d, zeros16, DC)
    return _tc_final(Tp, zs, dinv, b2[None, :])[:N]


# speedup vs baseline: 33.6640x; 2.3624x over previous
import functools

import jax
import jax.numpy as jnp
from jax import lax
from jax.experimental import pallas as pl
from jax.experimental.pallas import tpu as pltpu, tpu_sc as plsc

N = 10000
E = 320000
DF = 128
DH = 256
DC = 16
DQ = 32

NC = 2
NS = 16
ROW = 128
NROWS = 2560
EPAD = NROWS * ROW
NACC = 10240
NP = NACC
G = 4

_MESH = plsc.VectorSubcoreMesh(core_axis_name="c", subcore_axis_name="s")
_SC_PARAMS = pltpu.CompilerParams(use_tc_tiling_on_sc=False)


def _pipeline(srcv, dstv, table, acc, rows, sem_g, sem_s, nrows):
    npair = nrows // (2 * G)

    def gather(buf, j0):
        for b in range(G):
            pltpu.async_copy(table.at[srcv.at[j0 + b]], rows.at[buf, b], sem_g)

    def gather_wait(buf, j0):
        for b in range(G):
            pltpu.make_async_copy(table.at[srcv.at[j0 + b]], rows.at[buf, b],
                                  sem_g).wait()

    def scat(buf, j0):
        for b in range(G):
            pltpu.async_copy(rows.at[buf, b], acc.at[dstv.at[j0 + b]], sem_s,
                             add=True)

    def scat_wait(buf, j0):
        for b in range(G):
            pltpu.make_async_copy(rows.at[buf, b], acc.at[dstv.at[j0 + b]],
                                  sem_s).wait()

    gather(0, 0)

    def pair(t, carry):
        j0 = 2 * t * G
        j1 = j0 + G
        j2 = j0 + 2 * G
        gather_wait(0, j0)
        gather(1, j1)
        scat(0, j0)
        scat_wait(0, j0)
        gather_wait(1, j1)

        @pl.when(t + 1 < npair)
        def _():
            gather(0, j2)

        scat(1, j1)
        scat_wait(1, j1)
        return carry

    lax.fori_loop(0, npair, pair, 0)


def _sc_deg(dst2d, zeros16, ones16):
    @functools.partial(
        pl.kernel,
        out_type=jax.ShapeDtypeStruct((NC, NP, DC), jnp.float32),
        mesh=_MESH,
        compiler_params=_SC_PARAMS,
        scratch_types=[
            pltpu.VMEM((NROWS // (NC * NS), ROW), jnp.int32),
            pltpu.VMEM((ROW, DC), jnp.float32),
            pltpu.VMEM_SHARED((NACC, DC), jnp.float32),
        ],
    )
    def k(dst_hbm, z_hbm, ones_hbm, out_hbm, dstv, onev, acc):
        c = lax.axis_index("c")
        s = lax.axis_index("s")
        rpt = NROWS // (NC * NS)
        base = (c * NS + s) * rpt
        pltpu.sync_copy(z_hbm.at[pl.ds(s * (NACC // NS), NACC // NS)],
                        acc.at[pl.ds(s * (NACC // NS), NACC // NS)])
        pltpu.sync_copy(dst_hbm.at[pl.ds(base, rpt)], dstv)
        pltpu.sync_copy(ones_hbm, onev)
        plsc.subcore_barrier()

        def step(j, carry):
            pltpu.sync_copy(onev, acc.at[dstv.at[j]], add=True)
            return carry

        lax.fori_loop(0, rpt, step, 0)
        plsc.subcore_barrier()
        pltpu.sync_copy(acc.at[pl.ds(s * (NP // NS), NP // NS)],
                        out_hbm.at[c, pl.ds(s * (NP // NS), NP // NS)])

    return k(dst2d, zeros16, ones16)


def _sc_agg128(xs4, src2d, dst2d, zeros32):
    rpt = NROWS // NS

    @functools.partial(
        pl.kernel,
        out_type=jax.ShapeDtypeStruct((2 * NC, NP, DQ), jnp.float32),
        mesh=_MESH,
        compiler_params=_SC_PARAMS,
        scratch_types=[
            pltpu.VMEM((rpt, ROW), jnp.int32),
            pltpu.VMEM((rpt, ROW), jnp.int32),
            pltpu.VMEM((2, G, ROW, DQ), jnp.float32),
            pltpu.VMEM_SHARED((NACC, DQ), jnp.float32),
            pltpu.VMEM_SHARED((NACC, DQ), jnp.float32),
            pltpu.SemaphoreType.DMA,
            pltpu.SemaphoreType.DMA,
        ],
    )
    def k(xs_hbm, src_hbm, dst_hbm, z_hbm, out_hbm,
          srcv, dstv, rows, tab, acc, sem_g, sem_s):
        c = lax.axis_index("c")
        s = lax.axis_index("s")
        nsub = NACC // NS
        pltpu.sync_copy(src_hbm.at[pl.ds(s * rpt, rpt)], srcv)
        pltpu.sync_copy(dst_hbm.at[pl.ds(s * rpt, rpt)], dstv)
        for q in range(2):
            qi = q * NC + c
            pltpu.sync_copy(z_hbm.at[pl.ds(s * nsub, nsub)],
                            acc.at[pl.ds(s * nsub, nsub)])
            pltpu.sync_copy(xs_hbm.at[qi, pl.ds(s * nsub, nsub)],
                            tab.at[pl.ds(s * nsub, nsub)])
            plsc.subcore_barrier()
            _pipeline(srcv, dstv, tab, acc, rows, sem_g, sem_s, rpt)
            plsc.subcore_barrier()
            pltpu.sync_copy(acc.at[pl.ds(s * nsub, nsub)],
                            out_hbm.at[qi, pl.ds(s * nsub, nsub)])

    return k(xs4, src2d, dst2d, zeros32)


def _sc_agg16(zs, src2d, dst2d, zeros16):
    rpt = NROWS // (NC * NS)

    @functools.partial(
        pl.kernel,
        out_type=jax.ShapeDtypeStruct((NC, NP, DC), jnp.float32),
        mesh=_MESH,
        compiler_params=_SC_PARAMS,
        scratch_types=[
            pltpu.VMEM((rpt, ROW), jnp.int32),
            pltpu.VMEM((rpt, ROW), jnp.int32),
            pltpu.VMEM((2, G, ROW, DC), jnp.float32),
            pltpu.VMEM_SHARED((NACC, DC), jnp.float32),
            pltpu.VMEM_SHARED((NACC, DC), jnp.float32),
            pltpu.SemaphoreType.DMA,
            pltpu.SemaphoreType.DMA,
        ],
    )
    def k(zs_hbm, src_hbm, dst_hbm, z_hbm, out_hbm,
          srcv, dstv, rows, tab, acc, sem_g, sem_s):
        c = lax.axis_index("c")
        s = lax.axis_index("s")
        nsub = NACC // NS
        base = (c * NS + s) * rpt
        pltpu.sync_copy(z_hbm.at[pl.ds(s * nsub, nsub)],
                        acc.at[pl.ds(s * nsub, nsub)])
        pltpu.sync_copy(zs_hbm.at[pl.ds(s * nsub, nsub)],
                        tab.at[pl.ds(s * nsub, nsub)])
        pltpu.sync_copy(src_hbm.at[pl.ds(base, rpt)], srcv)
        pltpu.sync_copy(dst_hbm.at[pl.ds(base, rpt)], dstv)
        plsc.subcore_barrier()
        _pipeline(srcv, dstv, tab, acc, rows, sem_g, sem_s, rpt)
        plsc.subcore_barrier()
        pltpu.sync_copy(acc.at[pl.ds(s * nsub, nsub)],
                        out_hbm.at[c, pl.ds(s * nsub, nsub)])

    return k(zs, src2d, dst2d, zeros16)


_BLK = 1024
_GRID = NP // _BLK


def _tc_prep(degp, x):
    def body(degp_ref, x_ref, dinv_ref, xs_ref):
        deg = degp_ref[0, :, 0:1] + degp_ref[1, :, 0:1] + 1.0
        dinv = lax.rsqrt(deg)
        dinv_ref[...] = dinv
        for qi in range(4):
            xs_ref[qi] = x_ref[:, qi * DQ : (qi + 1) * DQ] * dinv

    return pl.pallas_call(
        body,
        grid=(_GRID,),
        in_specs=[
            pl.BlockSpec((NC, _BLK, DC), lambda i: (0, i, 0)),
            pl.BlockSpec((_BLK, DF), lambda i: (i, 0)),
        ],
        out_specs=[
            pl.BlockSpec((_BLK, 1), lambda i: (i, 0)),
            pl.BlockSpec((4, _BLK, DQ), lambda i: (0, i, 0)),
        ],
        out_shape=[
            jax.ShapeDtypeStruct((NP, 1), jnp.float32),
            jax.ShapeDtypeStruct((4, NP, DQ), jnp.float32),
        ],
    )(degp, x)


def _tc_mid(Sp, xs4, dinv, W1, b1, W2):
    def body(S_ref, xs_ref, dinv_ref, W1_ref, b1_ref, W2_ref, zs_ref):
        dinv = dinv_ref[...]
        parts = [S_ref[qi] + xs_ref[qi] for qi in range(4)]
        agg = jnp.concatenate(parts, axis=1) * dinv
        h = jnp.dot(agg, W1_ref[...], preferred_element_type=jnp.float32)
        h = jnp.maximum(h + b1_ref[...], 0.0)
        z = jnp.dot(h, W2_ref[...], preferred_element_type=jnp.float32)
        zs_ref[...] = z * dinv

    return pl.pallas_call(
        body,
        grid=(_GRID,),
        in_specs=[
            pl.BlockSpec((2 * NC, _BLK, DQ), lambda i: (0, i, 0)),
            pl.BlockSpec((2 * NC, _BLK, DQ), lambda i: (0, i, 0)),
            pl.BlockSpec((_BLK, 1), lambda i: (i, 0)),
            pl.BlockSpec((DF, DH), lambda i: (0, 0)),
            pl.BlockSpec((1, DH), lambda i: (0, 0)),
            pl.BlockSpec((DH, DC), lambda i: (0, 0)),
        ],
        out_specs=pl.BlockSpec((_BLK, DC), lambda i: (i, 0)),
        out_shape=jax.ShapeDtypeStruct((NP, DC), jnp.float32),
    )(Sp, xs4, dinv, W1, b1, W2)


def _tc_final(Tp, zs, dinv, b2):
    def body(T_ref, zs_ref, dinv_ref, b2_ref, out_ref):
        t = (T_ref[0] + T_ref[1] + zs_ref[...]) * dinv_ref[...] + b2_ref[...]
        m = jnp.max(t, axis=1, keepdims=True)
        e = jnp.exp(t - m)
        lse = jnp.log(jnp.sum(e, axis=1, keepdims=True))
        out_ref[...] = t - m - lse

    return pl.pallas_call(
        body,
        grid=(_GRID,),
        in_specs=[
            pl.BlockSpec((NC, _BLK, DC), lambda i: (0, i, 0)),
            pl.BlockSpec((_BLK, DC), lambda i: (i, 0)),
            pl.BlockSpec((_BLK, 1), lambda i: (i, 0)),
            pl.BlockSpec((1, DC), lambda i: (0, 0)),
        ],
        out_specs=pl.BlockSpec((_BLK, DC), lambda i: (i, 0)),
        out_shape=jax.ShapeDtypeStruct((NP, DC), jnp.float32),
    )(Tp, zs, dinv, b2)


def kernel(x, edge_idx, W1, b1, W2, b2):
    src = edge_idx[0]
    dst = edge_idx[1]
    pad = EPAD - E
    src2d = jnp.concatenate([src, jnp.zeros((pad,), jnp.int32)]).reshape(-1, ROW)
    dst2d = jnp.concatenate([dst, jnp.full((pad,), N, jnp.int32)]).reshape(-1, ROW)

    zeros16 = jnp.zeros((NACC, DC), jnp.float32)
    zeros32 = jnp.zeros((NACC, DQ), jnp.float32)
    ones16 = jnp.ones((ROW, DC), jnp.float32)

    xp = jnp.pad(x, ((0, NP - N), (0, 0)))
    degp = _sc_deg(dst2d, zeros16, ones16)
    dinv, xs4 = _tc_prep(degp, xp)
    Sp = _sc_agg128(xs4, src2d, dst2d, zeros32)
    zs = _tc_mid(Sp, xs4, dinv, W1, b1[None, :], W2)
    Tp = _sc_agg16(zs, src2d, dst2d, zeros16)
    return _tc_final(Tp, zs, dinv, b2[None, :])[:N]


# speedup vs baseline: 34.0226x; 1.0107x over previous
import functools

import jax
import jax.numpy as jnp
from jax import lax
from jax.experimental import pallas as pl
from jax.experimental.pallas import tpu as pltpu, tpu_sc as plsc

N = 10000
E = 320000
DF = 128
DH = 256
DC = 16
DQ = 32

NC = 2
NS = 16
ROW = 128
NROWS = 2560
EPAD = NROWS * ROW
NACC = 10240
NP = NACC
G = 5

_MESH = plsc.VectorSubcoreMesh(core_axis_name="c", subcore_axis_name="s")
_SC_PARAMS = pltpu.CompilerParams(use_tc_tiling_on_sc=False)


def _pipeline(srcv, dstv, table, acc, rows, sem_g, sem_s, nrows):
    npair = nrows // (2 * G)

    def gather(buf, j0):
        for b in range(G):
            pltpu.async_copy(table.at[srcv.at[j0 + b]], rows.at[buf, b], sem_g)

    def gather_wait(buf, j0):
        for b in range(G):
            pltpu.make_async_copy(table.at[srcv.at[j0 + b]], rows.at[buf, b],
                                  sem_g).wait()

    def scat(buf, j0):
        for b in range(G):
            pltpu.async_copy(rows.at[buf, b], acc.at[dstv.at[j0 + b]], sem_s,
                             add=True)

    def scat_wait(buf, j0):
        for b in range(G):
            pltpu.make_async_copy(rows.at[buf, b], acc.at[dstv.at[j0 + b]],
                                  sem_s).wait()

    gather(0, 0)

    def pair(t, carry):
        j0 = 2 * t * G
        j1 = j0 + G
        j2 = j0 + 2 * G
        gather_wait(0, j0)
        gather(1, j1)
        scat(0, j0)
        scat_wait(0, j0)
        gather_wait(1, j1)

        @pl.when(t + 1 < npair)
        def _():
            gather(0, j2)

        scat(1, j1)
        scat_wait(1, j1)
        return carry

    lax.fori_loop(0, npair, pair, 0)


def _sc_deg(dst2d, zeros16, ones16):
    @functools.partial(
        pl.kernel,
        out_type=jax.ShapeDtypeStruct((NC, NP, DC), jnp.float32),
        mesh=_MESH,
        compiler_params=_SC_PARAMS,
        scratch_types=[
            pltpu.VMEM((NROWS // (NC * NS), ROW), jnp.int32),
            pltpu.VMEM((ROW, DC), jnp.float32),
            pltpu.VMEM_SHARED((NACC, DC), jnp.float32),
        ],
    )
    def k(dst_hbm, z_hbm, ones_hbm, out_hbm, dstv, onev, acc):
        c = lax.axis_index("c")
        s = lax.axis_index("s")
        rpt = NROWS // (NC * NS)
        base = (c * NS + s) * rpt
        pltpu.sync_copy(z_hbm.at[pl.ds(s * (NACC // NS), NACC // NS)],
                        acc.at[pl.ds(s * (NACC // NS), NACC // NS)])
        pltpu.sync_copy(dst_hbm.at[pl.ds(base, rpt)], dstv)
        pltpu.sync_copy(ones_hbm, onev)
        plsc.subcore_barrier()

        def step(j, carry):
            pltpu.sync_copy(onev, acc.at[dstv.at[j]], add=True)
            return carry

        lax.fori_loop(0, rpt, step, 0)
        plsc.subcore_barrier()
        pltpu.sync_copy(acc.at[pl.ds(s * (NP // NS), NP // NS)],
                        out_hbm.at[c, pl.ds(s * (NP // NS), NP // NS)])

    return k(dst2d, zeros16, ones16)


def _sc_agg128(xs4, src2d, dst2d, zeros32):
    rpt = NROWS // NS

    @functools.partial(
        pl.kernel,
        out_type=jax.ShapeDtypeStruct((2 * NC, NP, DQ), jnp.float32),
        mesh=_MESH,
        compiler_params=_SC_PARAMS,
        scratch_types=[
            pltpu.VMEM((rpt, ROW), jnp.int32),
            pltpu.VMEM((rpt, ROW), jnp.int32),
            pltpu.VMEM((2, G, ROW, DQ), jnp.float32),
            pltpu.VMEM_SHARED((NACC, DQ), jnp.float32),
            pltpu.VMEM_SHARED((NACC, DQ), jnp.float32),
            pltpu.SemaphoreType.DMA,
            pltpu.SemaphoreType.DMA,
        ],
    )
    def k(xs_hbm, src_hbm, dst_hbm, z_hbm, out_hbm,
          srcv, dstv, rows, tab, acc, sem_g, sem_s):
        c = lax.axis_index("c")
        s = lax.axis_index("s")
        nsub = NACC // NS
        pltpu.sync_copy(src_hbm.at[pl.ds(s * rpt, rpt)], srcv)
        pltpu.sync_copy(dst_hbm.at[pl.ds(s * rpt, rpt)], dstv)
        for q in range(2):
            qi = q * NC + c
            pltpu.sync_copy(z_hbm.at[pl.ds(s * nsub, nsub)],
                            acc.at[pl.ds(s * nsub, nsub)])
            pltpu.sync_copy(xs_hbm.at[qi, pl.ds(s * nsub, nsub)],
                            tab.at[pl.ds(s * nsub, nsub)])
            plsc.subcore_barrier()
            _pipeline(srcv, dstv, tab, acc, rows, sem_g, sem_s, rpt)
            plsc.subcore_barrier()
            pltpu.sync_copy(acc.at[pl.ds(s * nsub, nsub)],
                            out_hbm.at[qi, pl.ds(s * nsub, nsub)])

    return k(xs4, src2d, dst2d, zeros32)


def _sc_agg16(zs, src2d, dst2d, zeros16):
    rpt = NROWS // (NC * NS)

    @functools.partial(
        pl.kernel,
        out_type=jax.ShapeDtypeStruct((NC, NP, DC), jnp.float32),
        mesh=_MESH,
        compiler_params=_SC_PARAMS,
        scratch_types=[
            pltpu.VMEM((rpt, ROW), jnp.int32),
            pltpu.VMEM((rpt, ROW), jnp.int32),
            pltpu.VMEM((2, G, ROW, DC), jnp.float32),
            pltpu.VMEM_SHARED((NACC, DC), jnp.float32),
            pltpu.VMEM_SHARED((NACC, DC), jnp.float32),
            pltpu.SemaphoreType.DMA,
            pltpu.SemaphoreType.DMA,
        ],
    )
    def k(zs_hbm, src_hbm, dst_hbm, z_hbm, out_hbm,
          srcv, dstv, rows, tab, acc, sem_g, sem_s):
        c = lax.axis_index("c")
        s = lax.axis_index("s")
        nsub = NACC // NS
        base = (c * NS + s) * rpt
        pltpu.sync_copy(z_hbm.at[pl.ds(s * nsub, nsub)],
                        acc.at[pl.ds(s * nsub, nsub)])
        pltpu.sync_copy(zs_hbm.at[pl.ds(s * nsub, nsub)],
                        tab.at[pl.ds(s * nsub, nsub)])
        pltpu.sync_copy(src_hbm.at[pl.ds(base, rpt)], srcv)
        pltpu.sync_copy(dst_hbm.at[pl.ds(base, rpt)], dstv)
        plsc.subcore_barrier()
        _pipeline(srcv, dstv, tab, acc, rows, sem_g, sem_s, rpt)
        plsc.subcore_barrier()
        pltpu.sync_copy(acc.at[pl.ds(s * nsub, nsub)],
                        out_hbm.at[c, pl.ds(s * nsub, nsub)])

    return k(zs, src2d, dst2d, zeros16)


_BLK = 1024
_GRID = NP // _BLK


def _tc_prep(degp, x):
    def body(degp_ref, x_ref, dinv_ref, xs_ref):
        deg = degp_ref[0, :, 0:1] + degp_ref[1, :, 0:1] + 1.0
        dinv = lax.rsqrt(deg)
        dinv_ref[...] = dinv
        for qi in range(4):
            xs_ref[qi] = x_ref[:, qi * DQ : (qi + 1) * DQ] * dinv

    return pl.pallas_call(
        body,
        grid=(_GRID,),
        in_specs=[
            pl.BlockSpec((NC, _BLK, DC), lambda i: (0, i, 0)),
            pl.BlockSpec((_BLK, DF), lambda i: (i, 0)),
        ],
        out_specs=[
            pl.BlockSpec((_BLK, 1), lambda i: (i, 0)),
            pl.BlockSpec((4, _BLK, DQ), lambda i: (0, i, 0)),
        ],
        out_shape=[
            jax.ShapeDtypeStruct((NP, 1), jnp.float32),
            jax.ShapeDtypeStruct((4, NP, DQ), jnp.float32),
        ],
    )(degp, x)


def _tc_mid(Sp, xs4, dinv, W1, b1, W2):
    def body(S_ref, xs_ref, dinv_ref, W1_ref, b1_ref, W2_ref, zs_ref):
        dinv = dinv_ref[...]
        parts = [S_ref[qi] + xs_ref[qi] for qi in range(4)]
        agg = jnp.concatenate(parts, axis=1) * dinv
        h = jnp.dot(agg, W1_ref[...], preferred_element_type=jnp.float32)
        h = jnp.maximum(h + b1_ref[...], 0.0)
        z = jnp.dot(h, W2_ref[...], preferred_element_type=jnp.float32)
        zs_ref[...] = z * dinv

    return pl.pallas_call(
        body,
        grid=(_GRID,),
        in_specs=[
            pl.BlockSpec((2 * NC, _BLK, DQ), lambda i: (0, i, 0)),
            pl.BlockSpec((2 * NC, _BLK, DQ), lambda i: (0, i, 0)),
            pl.BlockSpec((_BLK, 1), lambda i: (i, 0)),
            pl.BlockSpec((DF, DH), lambda i: (0, 0)),
            pl.BlockSpec((1, DH), lambda i: (0, 0)),
            pl.BlockSpec((DH, DC), lambda i: (0, 0)),
        ],
        out_specs=pl.BlockSpec((_BLK, DC), lambda i: (i, 0)),
        out_shape=jax.ShapeDtypeStruct((NP, DC), jnp.float32),
    )(Sp, xs4, dinv, W1, b1, W2)


def _tc_final(Tp, zs, dinv, b2):
    def body(T_ref, zs_ref, dinv_ref, b2_ref, out_ref):
        t = (T_ref[0] + T_ref[1] + zs_ref[...]) * dinv_ref[...] + b2_ref[...]
        m = jnp.max(t, axis=1, keepdims=True)
        e = jnp.exp(t - m)
        lse = jnp.log(jnp.sum(e, axis=1, keepdims=True))
        out_ref[...] = t - m - lse

    return pl.pallas_call(
        body,
        grid=(_GRID,),
        in_specs=[
            pl.BlockSpec((NC, _BLK, DC), lambda i: (0, i, 0)),
            pl.BlockSpec((_BLK, DC), lambda i: (i, 0)),
            pl.BlockSpec((_BLK, 1), lambda i: (i, 0)),
            pl.BlockSpec((1, DC), lambda i: (0, 0)),
        ],
        out_specs=pl.BlockSpec((_BLK, DC), lambda i: (i, 0)),
        out_shape=jax.ShapeDtypeStruct((NP, DC), jnp.float32),
    )(Tp, zs, dinv, b2)


def kernel(x, edge_idx, W1, b1, W2, b2):
    src = edge_idx[0]
    dst = edge_idx[1]
    pad = EPAD - E
    src2d = jnp.concatenate([src, jnp.zeros((pad,), jnp.int32)]).reshape(-1, ROW)
    dst2d = jnp.concatenate([dst, jnp.full((pad,), N, jnp.int32)]).reshape(-1, ROW)

    zeros16 = jnp.zeros((NACC, DC), jnp.float32)
    zeros32 = jnp.zeros((NACC, DQ), jnp.float32)
    ones16 = jnp.ones((ROW, DC), jnp.float32)

    xp = jnp.pad(x, ((0, NP - N), (0, 0)))
    degp = _sc_deg(dst2d, zeros16, ones16)
    dinv, xs4 = _tc_prep(degp, xp)
    Sp = _sc_agg128(xs4, src2d, dst2d, zeros32)
    zs = _tc_mid(Sp, xs4, dinv, W1, b1[None, :], W2)
    Tp = _sc_agg16(zs, src2d, dst2d, zeros16)
    return _tc_final(Tp, zs, dinv, b2[None, :])[:N]


# speedup vs baseline: 35.7908x; 1.0520x over previous
import functools

import jax
import jax.numpy as jnp
from jax import lax
from jax.experimental import pallas as pl
from jax.experimental.pallas import tpu as pltpu, tpu_sc as plsc

N = 10000
E = 320000
DF = 128
DH = 256
DC = 16
DQ = 32

NC = 2
NS = 16
ROW = 128
NROWS = 2560
EPAD = NROWS * ROW
NACC = 10240
NP = NACC

_MESH = plsc.VectorSubcoreMesh(core_axis_name="c", subcore_axis_name="s")
_SC_PARAMS = pltpu.CompilerParams(use_tc_tiling_on_sc=False)


def _pipeline(srcv, dstv, table, acc, rows, sem_g, sem_s, nrows):
    nbuf = rows.shape[0]
    gr = rows.shape[1]
    la = nbuf - 1
    ngrp = nrows // gr

    def gfire(g, bi):
        for b in range(gr):
            pltpu.async_copy(table.at[srcv.at[g * gr + b]], rows.at[bi, b],
                             sem_g)

    def gwait(g, bi):
        for b in range(gr):
            pltpu.make_async_copy(table.at[srcv.at[g * gr + b]],
                                  rows.at[bi, b], sem_g).wait()

    def sfire(g, bi):
        for b in range(gr):
            pltpu.async_copy(rows.at[bi, b], acc.at[dstv.at[g * gr + b]],
                             sem_s, add=True)

    def swait(g, bi):
        for b in range(gr):
            pltpu.make_async_copy(rows.at[bi, b], acc.at[dstv.at[g * gr + b]],
                                  sem_s).wait()

    for k in range(la):
        gfire(k, k)

    def body(g, carry):
        bi = lax.rem(g, nbuf)
        gwait(g, bi)
        sfire(g, bi)
        nxt = g + la
        bn = lax.rem(nxt, nbuf)

        @pl.when(jnp.logical_and(nxt < ngrp, nxt >= nbuf))
        def _():
            swait(nxt - nbuf, bn)

        @pl.when(nxt < ngrp)
        def _():
            gfire(nxt, bn)

        return carry

    lax.fori_loop(0, ngrp, body, 0)
    for g in range(max(0, ngrp - nbuf), ngrp):
        swait(g, g % nbuf)


def _sc_deg(dst2d, zeros16, ones16):
    @functools.partial(
        pl.kernel,
        out_type=jax.ShapeDtypeStruct((NC, NP, DC), jnp.float32),
        mesh=_MESH,
        compiler_params=_SC_PARAMS,
        scratch_types=[
            pltpu.VMEM((NROWS // (NC * NS), ROW), jnp.int32),
            pltpu.VMEM((ROW, DC), jnp.float32),
            pltpu.VMEM_SHARED((NACC, DC), jnp.float32),
            pltpu.SemaphoreType.DMA,
        ],
    )
    def k(dst_hbm, z_hbm, ones_hbm, out_hbm, dstv, onev, acc, sem):
        c = lax.axis_index("c")
        s = lax.axis_index("s")
        rpt = NROWS // (NC * NS)
        base = (c * NS + s) * rpt
        pltpu.sync_copy(z_hbm.at[pl.ds(s * (NACC // NS), NACC // NS)],
                        acc.at[pl.ds(s * (NACC // NS), NACC // NS)])
        pltpu.sync_copy(dst_hbm.at[pl.ds(base, rpt)], dstv)
        pltpu.sync_copy(ones_hbm, onev)
        plsc.subcore_barrier()

        def step(t, carry):
            for b in range(8):
                pltpu.async_copy(onev, acc.at[dstv.at[t * 8 + b]], sem,
                                 add=True)
            for b in range(8):
                pltpu.make_async_copy(onev, acc.at[dstv.at[t * 8 + b]],
                                      sem).wait()
            return carry

        lax.fori_loop(0, rpt // 8, step, 0)
        plsc.subcore_barrier()
        pltpu.sync_copy(acc.at[pl.ds(s * (NP // NS), NP // NS)],
                        out_hbm.at[c, pl.ds(s * (NP // NS), NP // NS)])

    return k(dst2d, zeros16, ones16)


def _sc_agg128(xs4, src2d, dst2d, zeros32):
    rpt = NROWS // NS

    @functools.partial(
        pl.kernel,
        out_type=jax.ShapeDtypeStruct((2 * NC, NP, DQ), jnp.float32),
        mesh=_MESH,
        compiler_params=_SC_PARAMS,
        scratch_types=[
            pltpu.VMEM((rpt, ROW), jnp.int32),
            pltpu.VMEM((rpt, ROW), jnp.int32),
            pltpu.VMEM((4, 2, ROW, DQ), jnp.float32),
            pltpu.VMEM_SHARED((NACC, DQ), jnp.float32),
            pltpu.VMEM_SHARED((NACC, DQ), jnp.float32),
            pltpu.SemaphoreType.DMA,
            pltpu.SemaphoreType.DMA,
        ],
    )
    def k(xs_hbm, src_hbm, dst_hbm, z_hbm, out_hbm,
          srcv, dstv, rows, tab, acc, sem_g, sem_s):
        c = lax.axis_index("c")
        s = lax.axis_index("s")
        nsub = NACC // NS
        pltpu.sync_copy(src_hbm.at[pl.ds(s * rpt, rpt)], srcv)
        pltpu.sync_copy(dst_hbm.at[pl.ds(s * rpt, rpt)], dstv)
        for q in range(2):
            qi = q * NC + c
            pltpu.sync_copy(z_hbm.at[pl.ds(s * nsub, nsub)],
                            acc.at[pl.ds(s * nsub, nsub)])
            pltpu.sync_copy(xs_hbm.at[qi, pl.ds(s * nsub, nsub)],
                            tab.at[pl.ds(s * nsub, nsub)])
            plsc.subcore_barrier()
            _pipeline(srcv, dstv, tab, acc, rows, sem_g, sem_s, rpt)
            plsc.subcore_barrier()
            pltpu.sync_copy(acc.at[pl.ds(s * nsub, nsub)],
                            out_hbm.at[qi, pl.ds(s * nsub, nsub)])

    return k(xs4, src2d, dst2d, zeros32)


def _sc_agg16(zs, src2d, dst2d, zeros16):
    rpt = NROWS // (NC * NS)

    @functools.partial(
        pl.kernel,
        out_type=jax.ShapeDtypeStruct((NC, NP, DC), jnp.float32),
        mesh=_MESH,
        compiler_params=_SC_PARAMS,
        scratch_types=[
            pltpu.VMEM((rpt, ROW), jnp.int32),
            pltpu.VMEM((rpt, ROW), jnp.int32),
            pltpu.VMEM((4, 2, ROW, DC), jnp.float32),
            pltpu.VMEM_SHARED((NACC, DC), jnp.float32),
            pltpu.VMEM_SHARED((NACC, DC), jnp.float32),
            pltpu.SemaphoreType.DMA,
            pltpu.SemaphoreType.DMA,
        ],
    )
    def k(zs_hbm, src_hbm, dst_hbm, z_hbm, out_hbm,
          srcv, dstv, rows, tab, acc, sem_g, sem_s):
        c = lax.axis_index("c")
        s = lax.axis_index("s")
        nsub = NACC // NS
        base = (c * NS + s) * rpt
        pltpu.sync_copy(z_hbm.at[pl.ds(s * nsub, nsub)],
                        acc.at[pl.ds(s * nsub, nsub)])
        pltpu.sync_copy(zs_hbm.at[pl.ds(s * nsub, nsub)],
                        tab.at[pl.ds(s * nsub, nsub)])
        pltpu.sync_copy(src_hbm.at[pl.ds(base, rpt)], srcv)
        pltpu.sync_copy(dst_hbm.at[pl.ds(base, rpt)], dstv)
        plsc.subcore_barrier()
        _pipeline(srcv, dstv, tab, acc, rows, sem_g, sem_s, rpt)
        plsc.subcore_barrier()
        pltpu.sync_copy(acc.at[pl.ds(s * nsub, nsub)],
                        out_hbm.at[c, pl.ds(s * nsub, nsub)])

    return k(zs, src2d, dst2d, zeros16)


_BLK = 1024
_GRID = NP // _BLK


def _tc_prep(degp, x):
    def body(degp_ref, x_ref, dinv_ref, xs_ref):
        deg = degp_ref[0, :, 0:1] + degp_ref[1, :, 0:1] + 1.0
        dinv = lax.rsqrt(deg)
        dinv_ref[...] = dinv
        for qi in range(4):
            xs_ref[qi] = x_ref[:, qi * DQ : (qi + 1) * DQ] * dinv

    return pl.pallas_call(
        body,
        grid=(_GRID,),
        in_specs=[
            pl.BlockSpec((NC, _BLK, DC), lambda i: (0, i, 0)),
            pl.BlockSpec((_BLK, DF), lambda i: (i, 0)),
        ],
        out_specs=[
            pl.BlockSpec((_BLK, 1), lambda i: (i, 0)),
            pl.BlockSpec((4, _BLK, DQ), lambda i: (0, i, 0)),
        ],
        out_shape=[
            jax.ShapeDtypeStruct((NP, 1), jnp.float32),
            jax.ShapeDtypeStruct((4, NP, DQ), jnp.float32),
        ],
    )(degp, x)


def _tc_mid(Sp, xs4, dinv, W1, b1, W2):
    def body(S_ref, xs_ref, dinv_ref, W1_ref, b1_ref, W2_ref, zs_ref):
        dinv = dinv_ref[...]
        parts = [S_ref[qi] + xs_ref[qi] for qi in range(4)]
        agg = jnp.concatenate(parts, axis=1) * dinv
        h = jnp.dot(agg, W1_ref[...], preferred_element_type=jnp.float32)
        h = jnp.maximum(h + b1_ref[...], 0.0)
        z = jnp.dot(h, W2_ref[...], preferred_element_type=jnp.float32)
        zs_ref[...] = z * dinv

    return pl.pallas_call(
        body,
        grid=(_GRID,),
        in_specs=[
            pl.BlockSpec((2 * NC, _BLK, DQ), lambda i: (0, i, 0)),
            pl.BlockSpec((2 * NC, _BLK, DQ), lambda i: (0, i, 0)),
            pl.BlockSpec((_BLK, 1), lambda i: (i, 0)),
            pl.BlockSpec((DF, DH), lambda i: (0, 0)),
            pl.BlockSpec((1, DH), lambda i: (0, 0)),
            pl.BlockSpec((DH, DC), lambda i: (0, 0)),
        ],
        out_specs=pl.BlockSpec((_BLK, DC), lambda i: (i, 0)),
        out_shape=jax.ShapeDtypeStruct((NP, DC), jnp.float32),
    )(Sp, xs4, dinv, W1, b1, W2)


def _tc_final(Tp, zs, dinv, b2):
    def body(T_ref, zs_ref, dinv_ref, b2_ref, out_ref):
        t = (T_ref[0] + T_ref[1] + zs_ref[...]) * dinv_ref[...] + b2_ref[...]
        m = jnp.max(t, axis=1, keepdims=True)
        e = jnp.exp(t - m)
        lse = jnp.log(jnp.sum(e, axis=1, keepdims=True))
        out_ref[...] = t - m - lse

    return pl.pallas_call(
        body,
        grid=(_GRID,),
        in_specs=[
            pl.BlockSpec((NC, _BLK, DC), lambda i: (0, i, 0)),
            pl.BlockSpec((_BLK, DC), lambda i: (i, 0)),
            pl.BlockSpec((_BLK, 1), lambda i: (i, 0)),
            pl.BlockSpec((1, DC), lambda i: (0, 0)),
        ],
        out_specs=pl.BlockSpec((_BLK, DC), lambda i: (i, 0)),
        out_shape=jax.ShapeDtypeStruct((NP, DC), jnp.float32),
    )(Tp, zs, dinv, b2)


def kernel(x, edge_idx, W1, b1, W2, b2):
    src = edge_idx[0]
    dst = edge_idx[1]
    pad = EPAD - E
    src2d = jnp.concatenate([src, jnp.zeros((pad,), jnp.int32)]).reshape(-1, ROW)
    dst2d = jnp.concatenate([dst, jnp.full((pad,), N, jnp.int32)]).reshape(-1, ROW)

    zeros16 = jnp.zeros((NACC, DC), jnp.float32)
    zeros32 = jnp.zeros((NACC, DQ), jnp.float32)
    ones16 = jnp.ones((ROW, DC), jnp.float32)

    xp = jnp.pad(x, ((0, NP - N), (0, 0)))
    degp = _sc_deg(dst2d, zeros16, ones16)
    dinv, xs4 = _tc_prep(degp, xp)
    Sp = _sc_agg128(xs4, src2d, dst2d, zeros32)
    zs = _tc_mid(Sp, xs4, dinv, W1, b1[None, :], W2)
    Tp = _sc_agg16(zs, src2d, dst2d, zeros16)
    return _tc_final(Tp, zs, dinv, b2[None, :])[:N]


# speedup vs baseline: 35.8522x; 1.0017x over previous
import functools

import jax
import jax.numpy as jnp
from jax import lax
from jax.experimental import pallas as pl
from jax.experimental.pallas import tpu as pltpu, tpu_sc as plsc

N = 10000
E = 320000
DF = 128
DH = 256
DC = 16
DQ = 32

NC = 2
NS = 16
ROW = 128
NROWS = 2560
EPAD = NROWS * ROW
NACC = 10240
NP = NACC

_MESH = plsc.VectorSubcoreMesh(core_axis_name="c", subcore_axis_name="s")
_SC_PARAMS = pltpu.CompilerParams(use_tc_tiling_on_sc=False)


def _pipeline(srcv, dstv, table, acc, rows, sem_g, sem_s, nrows):
    nbuf = rows.shape[0]
    gr = rows.shape[1]
    la = nbuf - 1
    ngrp = nrows // gr

    def gfire(g, bi):
        for b in range(gr):
            pltpu.async_copy(table.at[srcv.at[g * gr + b]], rows.at[bi, b],
                             sem_g)

    def gwait(g, bi):
        for b in range(gr):
            pltpu.make_async_copy(table.at[srcv.at[g * gr + b]],
                                  rows.at[bi, b], sem_g).wait()

    def sfire(g, bi):
        for b in range(gr):
            pltpu.async_copy(rows.at[bi, b], acc.at[dstv.at[g * gr + b]],
                             sem_s, add=True)

    def swait(g, bi):
        for b in range(gr):
            pltpu.make_async_copy(rows.at[bi, b], acc.at[dstv.at[g * gr + b]],
                                  sem_s).wait()

    for k in range(la):
        gfire(k, k)

    def body(g, carry):
        bi = lax.rem(g, nbuf)
        gwait(g, bi)
        sfire(g, bi)
        nxt = g + la
        bn = lax.rem(nxt, nbuf)

        @pl.when(jnp.logical_and(nxt < ngrp, nxt >= nbuf))
        def _():
            swait(nxt - nbuf, bn)

        @pl.when(nxt < ngrp)
        def _():
            gfire(nxt, bn)

        return carry

    lax.fori_loop(0, ngrp, body, 0)
    for g in range(max(0, ngrp - nbuf), ngrp):
        swait(g, g % nbuf)


def _sc_deg(dst2d, zeros16, ones16):
    @functools.partial(
        pl.kernel,
        out_type=jax.ShapeDtypeStruct((NC, NP, DC), jnp.float32),
        mesh=_MESH,
        compiler_params=_SC_PARAMS,
        scratch_types=[
            pltpu.VMEM((NROWS // (NC * NS), ROW), jnp.int32),
            pltpu.VMEM((ROW, DC), jnp.float32),
            pltpu.VMEM_SHARED((NACC, DC), jnp.float32),
            pltpu.SemaphoreType.DMA,
        ],
    )
    def k(dst_hbm, z_hbm, ones_hbm, out_hbm, dstv, onev, acc, sem):
        c = lax.axis_index("c")
        s = lax.axis_index("s")
        rpt = NROWS // (NC * NS)
        base = (c * NS + s) * rpt
        pltpu.sync_copy(z_hbm.at[pl.ds(s * (NACC // NS), NACC // NS)],
                        acc.at[pl.ds(s * (NACC // NS), NACC // NS)])
        pltpu.sync_copy(dst_hbm.at[pl.ds(base, rpt)], dstv)
        pltpu.sync_copy(ones_hbm, onev)
        plsc.subcore_barrier()

        def step(t, carry):
            for b in range(8):
                pltpu.async_copy(onev, acc.at[dstv.at[t * 8 + b]], sem,
                                 add=True)
            for b in range(8):
                pltpu.make_async_copy(onev, acc.at[dstv.at[t * 8 + b]],
                                      sem).wait()
            return carry

        lax.fori_loop(0, rpt // 8, step, 0)
        plsc.subcore_barrier()
        pltpu.sync_copy(acc.at[pl.ds(s * (NP // NS), NP // NS)],
                        out_hbm.at[c, pl.ds(s * (NP // NS), NP // NS)])

    return k(dst2d, zeros16, ones16)


def _sc_agg128(xs4, src2d, dst2d, zeros32):
    rpt = NROWS // NS

    @functools.partial(
        pl.kernel,
        out_type=jax.ShapeDtypeStruct((2 * NC, NP, DQ), jnp.float32),
        mesh=_MESH,
        compiler_params=_SC_PARAMS,
        scratch_types=[
            pltpu.VMEM((rpt, ROW), jnp.int32),
            pltpu.VMEM((rpt, ROW), jnp.int32),
            pltpu.VMEM((5, 2, ROW, DQ), jnp.float32),
            pltpu.VMEM_SHARED((NACC, DQ), jnp.float32),
            pltpu.VMEM_SHARED((NACC, DQ), jnp.float32),
            pltpu.SemaphoreType.DMA,
            pltpu.SemaphoreType.DMA,
        ],
    )
    def k(xs_hbm, src_hbm, dst_hbm, z_hbm, out_hbm,
          srcv, dstv, rows, tab, acc, sem_g, sem_s):
        c = lax.axis_index("c")
        s = lax.axis_index("s")
        nsub = NACC // NS
        pltpu.sync_copy(src_hbm.at[pl.ds(s * rpt, rpt)], srcv)
        pltpu.sync_copy(dst_hbm.at[pl.ds(s * rpt, rpt)], dstv)
        for q in range(2):
            qi = q * NC + c
            pltpu.sync_copy(z_hbm.at[pl.ds(s * nsub, nsub)],
                            acc.at[pl.ds(s * nsub, nsub)])
            pltpu.sync_copy(xs_hbm.at[qi, pl.ds(s * nsub, nsub)],
                            tab.at[pl.ds(s * nsub, nsub)])
            plsc.subcore_barrier()
            _pipeline(srcv, dstv, tab, acc, rows, sem_g, sem_s, rpt)
            plsc.subcore_barrier()
            pltpu.sync_copy(acc.at[pl.ds(s * nsub, nsub)],
                            out_hbm.at[qi, pl.ds(s * nsub, nsub)])

    return k(xs4, src2d, dst2d, zeros32)


def _sc_agg16(zs, src2d, dst2d, zeros16):
    rpt = NROWS // (NC * NS)

    @functools.partial(
        pl.kernel,
        out_type=jax.ShapeDtypeStruct((NC, NP, DC), jnp.float32),
        mesh=_MESH,
        compiler_params=_SC_PARAMS,
        scratch_types=[
            pltpu.VMEM((rpt, ROW), jnp.int32),
            pltpu.VMEM((rpt, ROW), jnp.int32),
            pltpu.VMEM((5, 2, ROW, DC), jnp.float32),
            pltpu.VMEM_SHARED((NACC, DC), jnp.float32),
            pltpu.VMEM_SHARED((NACC, DC), jnp.float32),
            pltpu.SemaphoreType.DMA,
            pltpu.SemaphoreType.DMA,
        ],
    )
    def k(zs_hbm, src_hbm, dst_hbm, z_hbm, out_hbm,
          srcv, dstv, rows, tab, acc, sem_g, sem_s):
        c = lax.axis_index("c")
        s = lax.axis_index("s")
        nsub = NACC // NS
        base = (c * NS + s) * rpt
        pltpu.sync_copy(z_hbm.at[pl.ds(s * nsub, nsub)],
                        acc.at[pl.ds(s * nsub, nsub)])
        pltpu.sync_copy(zs_hbm.at[pl.ds(s * nsub, nsub)],
                        tab.at[pl.ds(s * nsub, nsub)])
        pltpu.sync_copy(src_hbm.at[pl.ds(base, rpt)], srcv)
        pltpu.sync_copy(dst_hbm.at[pl.ds(base, rpt)], dstv)
        plsc.subcore_barrier()
        _pipeline(srcv, dstv, tab, acc, rows, sem_g, sem_s, rpt)
        plsc.subcore_barrier()
        pltpu.sync_copy(acc.at[pl.ds(s * nsub, nsub)],
                        out_hbm.at[c, pl.ds(s * nsub, nsub)])

    return k(zs, src2d, dst2d, zeros16)


_BLK = 1024
_GRID = NP // _BLK


def _tc_prep(degp, x):
    def body(degp_ref, x_ref, dinv_ref, xs_ref):
        deg = degp_ref[0, :, 0:1] + degp_ref[1, :, 0:1] + 1.0
        dinv = lax.rsqrt(deg)
        dinv_ref[...] = dinv
        for qi in range(4):
            xs_ref[qi] = x_ref[:, qi * DQ : (qi + 1) * DQ] * dinv

    return pl.pallas_call(
        body,
        grid=(_GRID,),
        in_specs=[
            pl.BlockSpec((NC, _BLK, DC), lambda i: (0, i, 0)),
            pl.BlockSpec((_BLK, DF), lambda i: (i, 0)),
        ],
        out_specs=[
            pl.BlockSpec((_BLK, 1), lambda i: (i, 0)),
            pl.BlockSpec((4, _BLK, DQ), lambda i: (0, i, 0)),
        ],
        out_shape=[
            jax.ShapeDtypeStruct((NP, 1), jnp.float32),
            jax.ShapeDtypeStruct((4, NP, DQ), jnp.float32),
        ],
    )(degp, x)


def _tc_mid(Sp, xs4, dinv, W1, b1, W2):
    def body(S_ref, xs_ref, dinv_ref, W1_ref, b1_ref, W2_ref, zs_ref):
        dinv = dinv_ref[...]
        parts = [S_ref[qi] + xs_ref[qi] for qi in range(4)]
        agg = jnp.concatenate(parts, axis=1) * dinv
        h = jnp.dot(agg, W1_ref[...], preferred_element_type=jnp.float32)
        h = jnp.maximum(h + b1_ref[...], 0.0)
        z = jnp.dot(h, W2_ref[...], preferred_element_type=jnp.float32)
        zs_ref[...] = z * dinv

    return pl.pallas_call(
        body,
        grid=(_GRID,),
        in_specs=[
            pl.BlockSpec((2 * NC, _BLK, DQ), lambda i: (0, i, 0)),
            pl.BlockSpec((2 * NC, _BLK, DQ), lambda i: (0, i, 0)),
            pl.BlockSpec((_BLK, 1), lambda i: (i, 0)),
            pl.BlockSpec((DF, DH), lambda i: (0, 0)),
            pl.BlockSpec((1, DH), lambda i: (0, 0)),
            pl.BlockSpec((DH, DC), lambda i: (0, 0)),
        ],
        out_specs=pl.BlockSpec((_BLK, DC), lambda i: (i, 0)),
        out_shape=jax.ShapeDtypeStruct((NP, DC), jnp.float32),
    )(Sp, xs4, dinv, W1, b1, W2)


def _tc_final(Tp, zs, dinv, b2):
    def body(T_ref, zs_ref, dinv_ref, b2_ref, out_ref):
        t = (T_ref[0] + T_ref[1] + zs_ref[...]) * dinv_ref[...] + b2_ref[...]
        m = jnp.max(t, axis=1, keepdims=True)
        e = jnp.exp(t - m)
        lse = jnp.log(jnp.sum(e, axis=1, keepdims=True))
        out_ref[...] = t - m - lse

    return pl.pallas_call(
        body,
        grid=(_GRID,),
        in_specs=[
            pl.BlockSpec((NC, _BLK, DC), lambda i: (0, i, 0)),
            pl.BlockSpec((_BLK, DC), lambda i: (i, 0)),
            pl.BlockSpec((_BLK, 1), lambda i: (i, 0)),
            pl.BlockSpec((1, DC), lambda i: (0, 0)),
        ],
        out_specs=pl.BlockSpec((_BLK, DC), lambda i: (i, 0)),
        out_shape=jax.ShapeDtypeStruct((NP, DC), jnp.float32),
    )(Tp, zs, dinv, b2)


def kernel(x, edge_idx, W1, b1, W2, b2):
    src = edge_idx[0]
    dst = edge_idx[1]
    pad = EPAD - E
    src2d = jnp.concatenate([src, jnp.zeros((pad,), jnp.int32)]).reshape(-1, ROW)
    dst2d = jnp.concatenate([dst, jnp.full((pad,), N, jnp.int32)]).reshape(-1, ROW)

    zeros16 = jnp.zeros((NACC, DC), jnp.float32)
    zeros32 = jnp.zeros((NACC, DQ), jnp.float32)
    ones16 = jnp.ones((ROW, DC), jnp.float32)

    xp = jnp.pad(x, ((0, NP - N), (0, 0)))
    degp = _sc_deg(dst2d, zeros16, ones16)
    dinv, xs4 = _tc_prep(degp, xp)
    Sp = _sc_agg128(xs4, src2d, dst2d, zeros32)
    zs = _tc_mid(Sp, xs4, dinv, W1, b1[None, :], W2)
    Tp = _sc_agg16(zs, src2d, dst2d, zeros16)
    return _tc_final(Tp, zs, dinv, b2[None, :])[:N]
